# Initial kernel scaffold; baseline (speedup 1.0000x reference)
import jax
import jax.numpy as jnp
from jax.experimental import pallas as pl


def kernel(x_s, edge_index_s, edge_attr_s, batch_s, x_t, edge_index_t, edge_attr_t, batch_t, W1_self, W1_msg, W1_edge, b1, W2_self, W2_msg, W2_edge, b2, M1, mb1, M2, mb2):
    raise NotImplementedError("write your pallas kernel here")


# speedup vs baseline: 1.5825x; 1.5825x over previous
import functools

import jax
import jax.numpy as jnp
from jax.experimental import pallas as pl
from jax.experimental.pallas import tpu as pltpu

N = 1024
E = 16384
EB = 1024
NBLK = E // EB
CH = 256
NCH = N // CH

_f32 = jnp.float32
_bf16 = jnp.bfloat16


def _bdot(a, b, dims=None):
    if dims is None:
        dims = (((a.ndim - 1,), (0,)), ((), ()))
    return jax.lax.dot_general(a.astype(_bf16), b.astype(_bf16), dims,
                               preferred_element_type=_f32)


def _split3(v):
    hi = v.astype(_bf16)
    mid = (v - hi.astype(_f32)).astype(_bf16)
    lo = (v - hi.astype(_f32) - mid.astype(_f32)).astype(_bf16)
    return hi, mid, lo


def _dot3(ab, parts):
    d = lambda x: jax.lax.dot_general(ab, x, (((1,), (0,)), ((), ())),
                                      preferred_element_type=_f32)
    return d(parts[0]) + d(parts[1]) + d(parts[2])


def _adj_build_kernel(dst_ref, src_ref, ea_blk_ref, W1e_ref, W2e_ref,
                      A_ref, ea1_ref, ea2_ref):
    j = pl.program_id(1)
    dst = dst_ref[0, 0, 0, :]
    src = src_ref[0, 0, 0, :]
    od = (jax.lax.broadcasted_iota(jnp.int32, (N, EB), 0)
          == dst[None, :]).astype(_bf16)
    os_ = (jax.lax.broadcasted_iota(jnp.int32, (EB, N), 1)
           == src[:, None]).astype(_bf16)
    contrib = jax.lax.dot_general(od, os_, (((1,), (0,)), ((), ())),
                                  preferred_element_type=_f32)
    eaW1 = _bdot(ea_blk_ref[0, 0], W1e_ref[...])
    eaW2 = _bdot(ea_blk_ref[0, 0], W2e_ref[...])
    ea1_contrib = _dot3(od, _split3(eaW1))
    ea2_contrib = _dot3(od, _split3(eaW2))

    @pl.when(j == 0)
    def _():
        A_ref[0] = contrib.astype(_bf16)
        ea1_ref[0] = ea1_contrib
        ea2_ref[0] = ea2_contrib

    @pl.when(j > 0)
    def _():
        A_ref[0] = (A_ref[0].astype(_f32) + contrib).astype(_bf16)
        ea1_ref[0] += ea1_contrib
        ea2_ref[0] += ea2_contrib


def _build_adj(dst_all, src_all, ea_all, W1e, W2e):
    return pl.pallas_call(
        _adj_build_kernel,
        grid=(2, NBLK),
        in_specs=[
            pl.BlockSpec((1, 1, 1, EB), lambda g, j: (g, j, 0, 0)),
            pl.BlockSpec((1, 1, 1, EB), lambda g, j: (g, j, 0, 0)),
            pl.BlockSpec((1, 1, EB, 16), lambda g, j: (g, j, 0, 0)),
            pl.BlockSpec((16, 128), lambda g, j: (0, 0)),
            pl.BlockSpec((16, 16), lambda g, j: (0, 0)),
        ],
        out_specs=[
            pl.BlockSpec((1, N, N), lambda g, j: (g, 0, 0)),
            pl.BlockSpec((1, N, 128), lambda g, j: (g, 0, 0)),
            pl.BlockSpec((1, N, 16), lambda g, j: (g, 0, 0)),
        ],
        out_shape=[
            jax.ShapeDtypeStruct((2, N, N), _bf16),
            jax.ShapeDtypeStruct((2, N, 128), _f32),
            jax.ShapeDtypeStruct((2, N, 16), _f32),
        ],
    )(dst_all, src_all, ea_all, W1e, W2e)


def _gnn_kernel(x_ref, A_ref, ea_ref, Wself_ref, Wmsg_ref, b_ref, h_ref):
    c = pl.program_id(1)
    pre_parts = _split3(_bdot(x_ref[0], Wmsg_ref[...]))
    agg = _dot3(A_ref[0], pre_parts)
    xr = x_ref[0, pl.ds(c * CH, CH), :]
    h_ref[0] = jax.nn.relu(_bdot(xr, Wself_ref[...]) + agg
                           + ea_ref[0] + b_ref[...])


def _gnn_call(x_all, A, ea, Wself, Wmsg, b_row, width):
    return pl.pallas_call(
        _gnn_kernel,
        grid=(2, NCH),
        in_specs=[
            pl.BlockSpec((1, N, width), lambda g, c: (g, 0, 0)),
            pl.BlockSpec((1, CH, N), lambda g, c: (g, c, 0)),
            pl.BlockSpec((1, CH, width), lambda g, c: (g, c, 0)),
            pl.BlockSpec((width, width), lambda g, c: (0, 0)),
            pl.BlockSpec((width, width), lambda g, c: (0, 0)),
            pl.BlockSpec((1, width), lambda g, c: (0, 0)),
        ],
        out_specs=pl.BlockSpec((1, CH, width), lambda g, c: (g, c, 0)),
        out_shape=jax.ShapeDtypeStruct((2, N, width), _f32),
    )(x_all, A, ea, Wself, Wmsg, b_row)


def _shat_init_kernel(hs_ref, ht_ref, xs0_ref, xt0_ref, shat_ref):
    shat_ref[...] = (_bdot(hs_ref[0], ht_ref[0], (((1,), (1,)), ((), ())))
                     - 2.0 * (xs0_ref[...] * xt0_ref[...]))


def _shat_init(h, xs0, xt0):
    return pl.pallas_call(
        _shat_init_kernel,
        grid=(NCH,),
        in_specs=[
            pl.BlockSpec((1, CH, 128), lambda c: (0, c, 0)),
            pl.BlockSpec((1, N, 128), lambda c: (1, 0, 0)),
            pl.BlockSpec((CH, 1), lambda c: (c, 0)),
            pl.BlockSpec((1, N), lambda c: (0, 0)),
        ],
        out_specs=pl.BlockSpec((CH, N), lambda c: (c, 0)),
        out_shape=jax.ShapeDtypeStruct((N, N), _f32),
    )(h, h, xs0, xt0)


def _softmax_rt_kernel(shat_ref, r_ref, S_ref, rt_ref):
    c = pl.program_id(0)
    sh = shat_ref[...]
    m = jnp.max(sh, axis=-1, keepdims=True)
    e = jnp.exp(sh - m)
    S = e / jnp.sum(e, axis=-1, keepdims=True)
    S_ref[...] = S
    contrib = _bdot(S, r_ref[...], (((0,), (0,)), ((), ())))

    @pl.when(c == 0)
    def _():
        rt_ref[...] = contrib

    @pl.when(c > 0)
    def _():
        rt_ref[...] += contrib


def _softmax_rt(shat, r_step):
    return pl.pallas_call(
        _softmax_rt_kernel,
        grid=(NCH,),
        in_specs=[
            pl.BlockSpec((CH, N), lambda c: (c, 0)),
            pl.BlockSpec((CH, 16), lambda c: (c, 0)),
        ],
        out_specs=[
            pl.BlockSpec((CH, N), lambda c: (c, 0)),
            pl.BlockSpec((N, 16), lambda c: (0, 0)),
        ],
        out_shape=[
            jax.ShapeDtypeStruct((N, N), _f32),
            jax.ShapeDtypeStruct((N, 16), _f32),
        ],
    )(shat, r_step)


def _softmax_kernel(shat_ref, S_ref):
    sh = shat_ref[...]
    m = jnp.max(sh, axis=-1, keepdims=True)
    e = jnp.exp(sh - m)
    S_ref[...] = e / jnp.sum(e, axis=-1, keepdims=True)


def _softmax_call(shat):
    return pl.pallas_call(
        _softmax_kernel,
        grid=(NCH,),
        in_specs=[pl.BlockSpec((CH, N), lambda c: (c, 0))],
        out_specs=pl.BlockSpec((CH, N), lambda c: (c, 0)),
        out_shape=jax.ShapeDtypeStruct((N, N), _f32),
    )(shat)


def _ot_kernel(ot_ref, OT_ref):
    parts = _split3(ot_ref[0])
    row_q = jax.lax.broadcasted_iota(jnp.int32, (128, N), 0)
    col_n = jax.lax.broadcasted_iota(jnp.int32, (128, N), 1)
    OT_ref[...] = jnp.concatenate(
        [_dot3((col_n == 8 * row_q + jj).astype(_bf16), parts)
         for jj in range(8)], axis=1)


def _ot_call(o):
    return pl.pallas_call(
        _ot_kernel,
        in_specs=[pl.BlockSpec((1, N, 16), lambda i: (1, 0, 0))],
        out_specs=pl.BlockSpec((128, 128), lambda i: (0, 0)),
        out_shape=jax.ShapeDtypeStruct((128, 128), _f32),
        grid=(1,),
    )(o)


def _update_kernel(shat_ref, o_ref, OT_ref, M1b_ref, mb1t_ref, M2b_ref,
                   mb2_ref, out_ref):
    macro = pl.program_id(0)
    o_s = o_ref[0]
    ostile = jnp.concatenate([o_s] * 8, axis=1)
    mb1t = mb1t_ref[...]
    parts = []
    for t in range(16):
        OTrow = OT_ref[pl.ds(macro * 16 + t, 1), :]
        D8 = ostile - OTrow
        h8 = jax.nn.relu(_bdot(D8, M1b_ref[...]) + mb1t)
        parts.append(_bdot(h8, M2b_ref[...]))
    out_ref[...] = shat_ref[...] + jnp.concatenate(parts, axis=1) \
        + mb2_ref[0, 0]


def _update_call(shat, o, OT, M1blk, mb1tile, M2blk, mb2_11):
    return pl.pallas_call(
        _update_kernel,
        grid=(8,),
        in_specs=[
            pl.BlockSpec((N, 128), lambda m: (0, m)),
            pl.BlockSpec((1, N, 16), lambda m: (0, 0, 0)),
            pl.BlockSpec((128, 128), lambda m: (0, 0)),
            pl.BlockSpec((128, 128), lambda m: (0, 0)),
            pl.BlockSpec((1, 128), lambda m: (0, 0)),
            pl.BlockSpec((128, 8), lambda m: (0, 0)),
            pl.BlockSpec((1, 1), lambda m: (0, 0)),
        ],
        out_specs=pl.BlockSpec((N, 128), lambda m: (0, m)),
        out_shape=jax.ShapeDtypeStruct((N, N), _f32),
    )(shat, o, OT, M1blk, mb1tile, M2blk, mb2_11)


def kernel(x_s, edge_index_s, edge_attr_s, batch_s, x_t, edge_index_t,
           edge_attr_t, batch_t, W1_self, W1_msg, W1_edge, b1,
           W2_self, W2_msg, W2_edge, b2, M1, mb1, M2, mb2):
    x_all = jnp.stack([x_s[0], x_t[0]])

    dst_all = jnp.stack([edge_index_s[1], edge_index_t[1]]).astype(jnp.int32)
    dst_all = dst_all.reshape(2, NBLK, 1, EB)
    src_all = jnp.stack([edge_index_s[0], edge_index_t[0]]).astype(jnp.int32)
    src_all = src_all.reshape(2, NBLK, 1, EB)
    ea_all = jnp.stack([edge_attr_s, edge_attr_t]).reshape(2, NBLK, EB, 16)

    A, ea1, ea2 = _build_adj(dst_all, src_all, ea_all, W1_edge, W2_edge)

    h = _gnn_call(x_all, A, ea1, W1_self, W1_msg, b1.reshape(1, -1), 128)
    shat = _shat_init(h, x_s[0, :, 0].reshape(N, 1), x_t[0, :, 0].reshape(1, N))

    rkey = jax.random.key(42)
    r = [jax.random.normal(jax.random.fold_in(rkey, step), (N, 16), _f32)
         for step in range(2)]

    eye8 = jnp.eye(8, dtype=_f32)
    M1blk = jnp.kron(eye8, M1)
    M2blk = jnp.kron(eye8, M2)
    mb1tile = jnp.tile(mb1, (8,)).reshape(1, 128)
    mb2_11 = mb2.reshape(1, 1)

    S0 = None
    for step in range(2):
        S, r_t = _softmax_rt(shat, r[step])
        if step == 0:
            S0 = S
        rst = jnp.stack([r[step], r_t])
        o = _gnn_call(rst, A, ea2, W2_self, W2_msg, b2.reshape(1, -1), 16)
        OT = _ot_call(o)
        shat = _update_call(shat, o, OT, M1blk, mb1tile, M2blk, mb2_11)

    SL = _softmax_call(shat)
    return (S0, SL)


# speedup vs baseline: 2.0770x; 1.3125x over previous
import functools

import jax
import jax.numpy as jnp
from jax.experimental import pallas as pl
from jax.experimental.pallas import tpu as pltpu

N = 1024
E = 16384
EB = 1024
NBLK = E // EB
CH = 256
NCH = N // CH

_f32 = jnp.float32
_bf16 = jnp.bfloat16


def _bdot(a, b, dims=None):
    if dims is None:
        dims = (((a.ndim - 1,), (0,)), ((), ()))
    return jax.lax.dot_general(a.astype(_bf16), b.astype(_bf16), dims,
                               preferred_element_type=_f32)


def _split3(v):
    hi = v.astype(_bf16)
    mid = (v - hi.astype(_f32)).astype(_bf16)
    lo = (v - hi.astype(_f32) - mid.astype(_f32)).astype(_bf16)
    return hi, mid, lo


def _dot3(ab, parts):
    d = lambda x: jax.lax.dot_general(ab, x, (((1,), (0,)), ((), ())),
                                      preferred_element_type=_f32)
    return d(parts[0]) + d(parts[1]) + d(parts[2])


def _adj_build_kernel(dst_ref, src_ref, ea_blk_ref, A_ref, sea_ref):
    j = pl.program_id(1)
    dst = dst_ref[0, 0, 0, :]
    src = src_ref[0, 0, 0, :]
    od = (jax.lax.broadcasted_iota(jnp.int32, (N, EB), 0)
          == dst[None, :]).astype(_bf16)
    os_ = (jax.lax.broadcasted_iota(jnp.int32, (EB, N), 1)
           == src[:, None]).astype(_bf16)
    contrib = jax.lax.dot_general(od, os_, (((1,), (0,)), ((), ())),
                                  preferred_element_type=_f32)
    sea_contrib = jax.lax.dot_general(
        od, ea_blk_ref[0, 0].astype(_bf16), (((1,), (0,)), ((), ())),
        preferred_element_type=_f32)

    @pl.when(j == 0)
    def _():
        A_ref[0] = contrib.astype(_bf16)
        sea_ref[0] = sea_contrib

    @pl.when(j > 0)
    def _():
        A_ref[0] = (A_ref[0].astype(_f32) + contrib).astype(_bf16)
        sea_ref[0] += sea_contrib


def _build_adj(dst_all, src_all, ea_all):
    return pl.pallas_call(
        _adj_build_kernel,
        grid=(2, NBLK),
        in_specs=[
            pl.BlockSpec((1, 1, 1, EB), lambda g, j: (g, j, 0, 0)),
            pl.BlockSpec((1, 1, 1, EB), lambda g, j: (g, j, 0, 0)),
            pl.BlockSpec((1, 1, EB, 16), lambda g, j: (g, j, 0, 0)),
        ],
        out_specs=[
            pl.BlockSpec((1, N, N), lambda g, j: (g, 0, 0)),
            pl.BlockSpec((1, N, 16), lambda g, j: (g, 0, 0)),
        ],
        out_shape=[
            jax.ShapeDtypeStruct((2, N, N), _bf16),
            jax.ShapeDtypeStruct((2, N, 16), _f32),
        ],
    )(dst_all, src_all, ea_all)


def _ea_kernel(sea_ref, W1e_ref, W2e_ref, ea1_ref, ea2_ref):
    parts = _split3(sea_ref[0])
    w1 = W1e_ref[...].astype(_bf16)
    w2 = W2e_ref[...].astype(_bf16)

    def d3(w):
        d = lambda p: jax.lax.dot_general(p, w, (((1,), (0,)), ((), ())),
                                          preferred_element_type=_f32)
        return d(parts[0]) + d(parts[1]) + d(parts[2])

    ea1_ref[0] = d3(w1)
    ea2_ref[0] = d3(w2)


def _ea_call(sea, W1e, W2e):
    return pl.pallas_call(
        _ea_kernel,
        grid=(2,),
        in_specs=[
            pl.BlockSpec((1, N, 16), lambda g: (g, 0, 0)),
            pl.BlockSpec((16, 128), lambda g: (0, 0)),
            pl.BlockSpec((16, 16), lambda g: (0, 0)),
        ],
        out_specs=[
            pl.BlockSpec((1, N, 128), lambda g: (g, 0, 0)),
            pl.BlockSpec((1, N, 16), lambda g: (g, 0, 0)),
        ],
        out_shape=[
            jax.ShapeDtypeStruct((2, N, 128), _f32),
            jax.ShapeDtypeStruct((2, N, 16), _f32),
        ],
    )(sea, W1e, W2e)


def _gnn_kernel(x_ref, A_ref, ea_ref, Wself_ref, Wmsg_ref, b_ref, h_ref):
    c = pl.program_id(1)
    pre_parts = _split3(_bdot(x_ref[0], Wmsg_ref[...]))
    agg = _dot3(A_ref[0], pre_parts)
    xr = x_ref[0, pl.ds(c * CH, CH), :]
    h_ref[0] = jax.nn.relu(_bdot(xr, Wself_ref[...]) + agg
                           + ea_ref[0] + b_ref[...])


def _gnn_call(x_all, A, ea, Wself, Wmsg, b_row, width):
    return pl.pallas_call(
        _gnn_kernel,
        grid=(2, NCH),
        in_specs=[
            pl.BlockSpec((1, N, width), lambda g, c: (g, 0, 0)),
            pl.BlockSpec((1, CH, N), lambda g, c: (g, c, 0)),
            pl.BlockSpec((1, CH, width), lambda g, c: (g, c, 0)),
            pl.BlockSpec((width, width), lambda g, c: (0, 0)),
            pl.BlockSpec((width, width), lambda g, c: (0, 0)),
            pl.BlockSpec((1, width), lambda g, c: (0, 0)),
        ],
        out_specs=pl.BlockSpec((1, CH, width), lambda g, c: (g, c, 0)),
        out_shape=jax.ShapeDtypeStruct((2, N, width), _f32),
    )(x_all, A, ea, Wself, Wmsg, b_row)


def _shat_init_kernel(hs_ref, ht_ref, xs0_ref, xt0_ref, shat_ref):
    shat_ref[...] = (_bdot(hs_ref[0], ht_ref[0], (((1,), (1,)), ((), ())))
                     - 2.0 * (xs0_ref[...] * xt0_ref[...]))


def _shat_init(h, xs0, xt0):
    return pl.pallas_call(
        _shat_init_kernel,
        grid=(NCH,),
        in_specs=[
            pl.BlockSpec((1, CH, 128), lambda c: (0, c, 0)),
            pl.BlockSpec((1, N, 128), lambda c: (1, 0, 0)),
            pl.BlockSpec((CH, 1), lambda c: (c, 0)),
            pl.BlockSpec((1, N), lambda c: (0, 0)),
        ],
        out_specs=pl.BlockSpec((CH, N), lambda c: (c, 0)),
        out_shape=jax.ShapeDtypeStruct((N, N), _f32),
    )(h, h, xs0, xt0)


def _softmax_rt_kernel(shat_ref, r_ref, S_ref, rt_ref):
    c = pl.program_id(0)
    sh = shat_ref[...]
    m = jnp.max(sh, axis=-1, keepdims=True)
    e = jnp.exp(sh - m)
    S = e / jnp.sum(e, axis=-1, keepdims=True)
    S_ref[...] = S
    contrib = _bdot(S, r_ref[...], (((0,), (0,)), ((), ())))

    @pl.when(c == 0)
    def _():
        rt_ref[...] = contrib

    @pl.when(c > 0)
    def _():
        rt_ref[...] += contrib


def _softmax_rt(shat, r_step):
    return pl.pallas_call(
        _softmax_rt_kernel,
        grid=(NCH,),
        in_specs=[
            pl.BlockSpec((CH, N), lambda c: (c, 0)),
            pl.BlockSpec((CH, 16), lambda c: (c, 0)),
        ],
        out_specs=[
            pl.BlockSpec((CH, N), lambda c: (c, 0)),
            pl.BlockSpec((N, 16), lambda c: (0, 0)),
        ],
        out_shape=[
            jax.ShapeDtypeStruct((N, N), _f32),
            jax.ShapeDtypeStruct((N, 16), _f32),
        ],
    )(shat, r_step)


def _softmax_kernel(shat_ref, S_ref):
    sh = shat_ref[...]
    m = jnp.max(sh, axis=-1, keepdims=True)
    e = jnp.exp(sh - m)
    S_ref[...] = e / jnp.sum(e, axis=-1, keepdims=True)


def _softmax_call(shat):
    return pl.pallas_call(
        _softmax_kernel,
        grid=(NCH,),
        in_specs=[pl.BlockSpec((CH, N), lambda c: (c, 0))],
        out_specs=pl.BlockSpec((CH, N), lambda c: (c, 0)),
        out_shape=jax.ShapeDtypeStruct((N, N), _f32),
    )(shat)


def _ot_kernel(ot_ref, OT_ref):
    parts = _split3(ot_ref[0])
    row_q = jax.lax.broadcasted_iota(jnp.int32, (128, N), 0)
    col_n = jax.lax.broadcasted_iota(jnp.int32, (128, N), 1)
    OT_ref[...] = jnp.concatenate(
        [_dot3((col_n == 8 * row_q + jj).astype(_bf16), parts)
         for jj in range(8)], axis=1)


def _ot_call(o):
    return pl.pallas_call(
        _ot_kernel,
        in_specs=[pl.BlockSpec((1, N, 16), lambda i: (1, 0, 0))],
        out_specs=pl.BlockSpec((128, 128), lambda i: (0, 0)),
        out_shape=jax.ShapeDtypeStruct((128, 128), _f32),
        grid=(1,),
    )(o)


def _update_kernel(shat_ref, o_ref, OT_ref, M1b_ref, mb1t_ref, M2b_ref,
                   mb2_ref, out_ref):
    macro = pl.program_id(0)
    o_s = o_ref[0]
    ostile = jnp.concatenate([o_s] * 8, axis=1)
    mb1t = mb1t_ref[...]
    parts = []
    for t in range(16):
        OTrow = OT_ref[pl.ds(macro * 16 + t, 1), :]
        D8 = ostile - OTrow
        h8 = jax.nn.relu(_bdot(D8, M1b_ref[...]) + mb1t)
        parts.append(_bdot(h8, M2b_ref[...]))
    out_ref[...] = shat_ref[...] + jnp.concatenate(parts, axis=1) \
        + mb2_ref[0, 0]


def _update_call(shat, o, OT, M1blk, mb1tile, M2blk, mb2_11):
    return pl.pallas_call(
        _update_kernel,
        grid=(8,),
        in_specs=[
            pl.BlockSpec((N, 128), lambda m: (0, m)),
            pl.BlockSpec((1, N, 16), lambda m: (0, 0, 0)),
            pl.BlockSpec((128, 128), lambda m: (0, 0)),
            pl.BlockSpec((128, 128), lambda m: (0, 0)),
            pl.BlockSpec((1, 128), lambda m: (0, 0)),
            pl.BlockSpec((128, 8), lambda m: (0, 0)),
            pl.BlockSpec((1, 1), lambda m: (0, 0)),
        ],
        out_specs=pl.BlockSpec((N, 128), lambda m: (0, m)),
        out_shape=jax.ShapeDtypeStruct((N, N), _f32),
    )(shat, o, OT, M1blk, mb1tile, M2blk, mb2_11)


def kernel(x_s, edge_index_s, edge_attr_s, batch_s, x_t, edge_index_t,
           edge_attr_t, batch_t, W1_self, W1_msg, W1_edge, b1,
           W2_self, W2_msg, W2_edge, b2, M1, mb1, M2, mb2):
    x_all = jnp.stack([x_s[0], x_t[0]])

    dst_all = jnp.stack([edge_index_s[1], edge_index_t[1]]).astype(jnp.int32)
    dst_all = dst_all.reshape(2, NBLK, 1, EB)
    src_all = jnp.stack([edge_index_s[0], edge_index_t[0]]).astype(jnp.int32)
    src_all = src_all.reshape(2, NBLK, 1, EB)
    ea_all = jnp.stack([edge_attr_s, edge_attr_t]).reshape(2, NBLK, EB, 16)

    A, sea = _build_adj(dst_all, src_all, ea_all)
    ea1, ea2 = _ea_call(sea, W1_edge, W2_edge)

    h = _gnn_call(x_all, A, ea1, W1_self, W1_msg, b1.reshape(1, -1), 128)
    shat = _shat_init(h, x_s[0, :, 0].reshape(N, 1), x_t[0, :, 0].reshape(1, N))

    rkey = jax.random.key(42)
    r = [jax.random.normal(jax.random.fold_in(rkey, step), (N, 16), _f32)
         for step in range(2)]

    eye8 = jnp.eye(8, dtype=_f32)
    M1blk = jnp.kron(eye8, M1)
    M2blk = jnp.kron(eye8, M2)
    mb1tile = jnp.tile(mb1, (8,)).reshape(1, 128)
    mb2_11 = mb2.reshape(1, 1)

    S0 = None
    for step in range(2):
        S, r_t = _softmax_rt(shat, r[step])
        if step == 0:
            S0 = S
        rst = jnp.stack([r[step], r_t])
        o = _gnn_call(rst, A, ea2, W2_self, W2_msg, b2.reshape(1, -1), 16)
        OT = _ot_call(o)
        shat = _update_call(shat, o, OT, M1blk, mb1tile, M2blk, mb2_11)

    SL = _softmax_call(shat)
    return (S0, SL)


# speedup vs baseline: 2.5874x; 1.2457x over previous
import functools

import jax
import jax.numpy as jnp
from jax.experimental import pallas as pl
from jax.experimental.pallas import tpu as pltpu
from jax.experimental.pallas import tpu_sc as plsc

N = 1024
E = 16384
EB = 1024
NBLK = E // EB
CH = 256
NCH = N // CH

_f32 = jnp.float32
_bf16 = jnp.bfloat16


def _bdot(a, b, dims=None):
    if dims is None:
        dims = (((a.ndim - 1,), (0,)), ((), ()))
    return jax.lax.dot_general(a.astype(_bf16), b.astype(_bf16), dims,
                               preferred_element_type=_f32)


def _split3(v):
    hi = v.astype(_bf16)
    mid = (v - hi.astype(_f32)).astype(_bf16)
    lo = (v - hi.astype(_f32) - mid.astype(_f32)).astype(_bf16)
    return hi, mid, lo


def _dot3(ab, parts):
    d = lambda x: jax.lax.dot_general(ab, x, (((1,), (0,)), ((), ())),
                                      preferred_element_type=_f32)
    return d(parts[0]) + d(parts[1]) + d(parts[2])


_ROWS = 64
_ECH = 1024
_CHK = 2048


def _sc_adj_kernel(dst_hbm, src_hbm, eat_hbm, A_hbm, sea_hbm,
                   dv, sv, ablk, eat, seap):
    g = jax.lax.axis_index("c")
    sid = jax.lax.axis_index("s")
    lo = sid * _ROWS

    zeros16 = jnp.zeros((16,), _f32)
    ones16 = jnp.ones((16,), _f32)

    def zrow(i, carry):
        ablk[pl.ds(i * 16, 16)] = zeros16
        return carry

    jax.lax.fori_loop(0, _ROWS * N // 16, zrow, 0)

    def zsea(i, carry):
        seap[pl.ds(i * 16, 16)] = zeros16
        return carry

    jax.lax.fori_loop(0, N * 16 // 16, zsea, 0)

    for c in range(E // _CHK):
        pltpu.sync_copy(dst_hbm.at[g, pl.ds(c * _CHK, _CHK)], dv)
        pltpu.sync_copy(src_hbm.at[g, pl.ds(c * _CHK, _CHK)], sv)

        def body(i, carry):
            d = dv[pl.ds(i * 16, 16)]
            s_ = sv[pl.ds(i * 16, 16)]
            mask = (d >= lo) & (d < lo + _ROWS)
            flat = jnp.where(mask, (d - lo) * N + s_, 0)
            plsc.addupdate_scatter(ablk, [flat], ones16, mask=mask)
            return carry

        jax.lax.fori_loop(0, _CHK // 16, body, 0)
    pltpu.sync_copy(ablk, A_hbm.at[g, pl.ds(lo * N, _ROWS * N)])

    pltpu.sync_copy(dst_hbm.at[g, pl.ds(sid * _ECH, _ECH)],
                    dv.at[pl.ds(0, _ECH)])
    pltpu.sync_copy(eat_hbm.at[g, :, pl.ds(sid * _ECH, _ECH)], eat)

    def ebody(v, carry):
        d16 = dv[pl.ds(v * 16, 16)]
        base = d16 * 16
        for m in range(16):
            vals = eat[m, pl.ds(v * 16, 16)]
            plsc.addupdate_scatter(seap, [base + m], vals)
        return carry

    jax.lax.fori_loop(0, _ECH // 16, ebody, 0)
    pltpu.sync_copy(seap, sea_hbm.at[g, sid])


def _build_adj_sc(dst, src, ea_t):
    mesh = plsc.VectorSubcoreMesh(core_axis_name="c", subcore_axis_name="s")
    run = functools.partial(
        pl.kernel,
        mesh=mesh,
        compiler_params=pltpu.CompilerParams(needs_layout_passes=False),
        out_type=[
            jax.ShapeDtypeStruct((2, N * N), _f32),
            jax.ShapeDtypeStruct((2, 16, N * 16), _f32),
        ],
        scratch_types=[
            pltpu.VMEM((_CHK,), jnp.int32),
            pltpu.VMEM((_CHK,), jnp.int32),
            pltpu.VMEM((_ROWS * N,), _f32),
            pltpu.VMEM((16, _ECH), _f32),
            pltpu.VMEM((N * 16,), _f32),
        ],
    )(_sc_adj_kernel)
    A_flat, sea_parts = run(dst, src, ea_t)
    return A_flat.reshape(2, N, N), sea_parts.reshape(2, 16, N, 16)


def _adj_build_kernel(dst_ref, src_ref, ea_blk_ref, A_ref, sea_ref):
    j = pl.program_id(1)
    dst = dst_ref[0, 0, 0, :]
    src = src_ref[0, 0, 0, :]
    od = (jax.lax.broadcasted_iota(jnp.int32, (N, EB), 0)
          == dst[None, :]).astype(_bf16)
    os_ = (jax.lax.broadcasted_iota(jnp.int32, (EB, N), 1)
           == src[:, None]).astype(_bf16)
    contrib = jax.lax.dot_general(od, os_, (((1,), (0,)), ((), ())),
                                  preferred_element_type=_f32)
    sea_contrib = jax.lax.dot_general(
        od, ea_blk_ref[0, 0].astype(_bf16), (((1,), (0,)), ((), ())),
        preferred_element_type=_f32)

    @pl.when(j == 0)
    def _():
        A_ref[0] = contrib.astype(_bf16)
        sea_ref[0] = sea_contrib

    @pl.when(j > 0)
    def _():
        A_ref[0] = (A_ref[0].astype(_f32) + contrib).astype(_bf16)
        sea_ref[0] += sea_contrib


def _build_adj(dst_all, src_all, ea_all):
    return pl.pallas_call(
        _adj_build_kernel,
        grid=(2, NBLK),
        in_specs=[
            pl.BlockSpec((1, 1, 1, EB), lambda g, j: (g, j, 0, 0)),
            pl.BlockSpec((1, 1, 1, EB), lambda g, j: (g, j, 0, 0)),
            pl.BlockSpec((1, 1, EB, 16), lambda g, j: (g, j, 0, 0)),
        ],
        out_specs=[
            pl.BlockSpec((1, N, N), lambda g, j: (g, 0, 0)),
            pl.BlockSpec((1, N, 16), lambda g, j: (g, 0, 0)),
        ],
        out_shape=[
            jax.ShapeDtypeStruct((2, N, N), _bf16),
            jax.ShapeDtypeStruct((2, N, 16), _f32),
        ],
    )(dst_all, src_all, ea_all)


def _ea_kernel(sea_ref, W1e_ref, W2e_ref, ea1_ref, ea2_ref):
    parts = _split3(jnp.sum(sea_ref[0], axis=0))
    w1 = W1e_ref[...].astype(_bf16)
    w2 = W2e_ref[...].astype(_bf16)

    def d3(w):
        d = lambda p: jax.lax.dot_general(p, w, (((1,), (0,)), ((), ())),
                                          preferred_element_type=_f32)
        return d(parts[0]) + d(parts[1]) + d(parts[2])

    ea1_ref[0] = d3(w1)
    ea2_ref[0] = d3(w2)


def _ea_call(sea, W1e, W2e):
    return pl.pallas_call(
        _ea_kernel,
        grid=(2,),
        in_specs=[
            pl.BlockSpec((1, 16, N, 16), lambda g: (g, 0, 0, 0)),
            pl.BlockSpec((16, 128), lambda g: (0, 0)),
            pl.BlockSpec((16, 16), lambda g: (0, 0)),
        ],
        out_specs=[
            pl.BlockSpec((1, N, 128), lambda g: (g, 0, 0)),
            pl.BlockSpec((1, N, 16), lambda g: (g, 0, 0)),
        ],
        out_shape=[
            jax.ShapeDtypeStruct((2, N, 128), _f32),
            jax.ShapeDtypeStruct((2, N, 16), _f32),
        ],
    )(sea, W1e, W2e)


def _gnn_kernel(x_ref, A_ref, ea_ref, Wself_ref, Wmsg_ref, b_ref, h_ref):
    c = pl.program_id(1)
    pre_parts = _split3(_bdot(x_ref[0], Wmsg_ref[...]))
    agg = _dot3(A_ref[0], pre_parts)
    xr = x_ref[0, pl.ds(c * CH, CH), :]
    h_ref[0] = jax.nn.relu(_bdot(xr, Wself_ref[...]) + agg
                           + ea_ref[0] + b_ref[...])


def _gnn_call(x_all, A, ea, Wself, Wmsg, b_row, width):
    return pl.pallas_call(
        _gnn_kernel,
        grid=(2, NCH),
        in_specs=[
            pl.BlockSpec((1, N, width), lambda g, c: (g, 0, 0)),
            pl.BlockSpec((1, CH, N), lambda g, c: (g, c, 0)),
            pl.BlockSpec((1, CH, width), lambda g, c: (g, c, 0)),
            pl.BlockSpec((width, width), lambda g, c: (0, 0)),
            pl.BlockSpec((width, width), lambda g, c: (0, 0)),
            pl.BlockSpec((1, width), lambda g, c: (0, 0)),
        ],
        out_specs=pl.BlockSpec((1, CH, width), lambda g, c: (g, c, 0)),
        out_shape=jax.ShapeDtypeStruct((2, N, width), _f32),
    )(x_all, A, ea, Wself, Wmsg, b_row)


def _shat_init_kernel(hs_ref, ht_ref, xs0_ref, xt0_ref, shat_ref):
    shat_ref[...] = (_bdot(hs_ref[0], ht_ref[0], (((1,), (1,)), ((), ())))
                     - 2.0 * (xs0_ref[...] * xt0_ref[...]))


def _shat_init(h, xs0, xt0):
    return pl.pallas_call(
        _shat_init_kernel,
        grid=(NCH,),
        in_specs=[
            pl.BlockSpec((1, CH, 128), lambda c: (0, c, 0)),
            pl.BlockSpec((1, N, 128), lambda c: (1, 0, 0)),
            pl.BlockSpec((CH, 1), lambda c: (c, 0)),
            pl.BlockSpec((1, N), lambda c: (0, 0)),
        ],
        out_specs=pl.BlockSpec((CH, N), lambda c: (c, 0)),
        out_shape=jax.ShapeDtypeStruct((N, N), _f32),
    )(h, h, xs0, xt0)


def _softmax_rt_kernel(shat_ref, r_ref, S_ref, rt_ref):
    c = pl.program_id(0)
    sh = shat_ref[...]
    m = jnp.max(sh, axis=-1, keepdims=True)
    e = jnp.exp(sh - m)
    S = e / jnp.sum(e, axis=-1, keepdims=True)
    S_ref[...] = S
    contrib = _bdot(S, r_ref[...], (((0,), (0,)), ((), ())))

    @pl.when(c == 0)
    def _():
        rt_ref[...] = contrib

    @pl.when(c > 0)
    def _():
        rt_ref[...] += contrib


def _softmax_rt(shat, r_step):
    return pl.pallas_call(
        _softmax_rt_kernel,
        grid=(NCH,),
        in_specs=[
            pl.BlockSpec((CH, N), lambda c: (c, 0)),
            pl.BlockSpec((CH, 16), lambda c: (c, 0)),
        ],
        out_specs=[
            pl.BlockSpec((CH, N), lambda c: (c, 0)),
            pl.BlockSpec((N, 16), lambda c: (0, 0)),
        ],
        out_shape=[
            jax.ShapeDtypeStruct((N, N), _f32),
            jax.ShapeDtypeStruct((N, 16), _f32),
        ],
    )(shat, r_step)


def _softmax_kernel(shat_ref, S_ref):
    sh = shat_ref[...]
    m = jnp.max(sh, axis=-1, keepdims=True)
    e = jnp.exp(sh - m)
    S_ref[...] = e / jnp.sum(e, axis=-1, keepdims=True)


def _softmax_call(shat):
    return pl.pallas_call(
        _softmax_kernel,
        grid=(NCH,),
        in_specs=[pl.BlockSpec((CH, N), lambda c: (c, 0))],
        out_specs=pl.BlockSpec((CH, N), lambda c: (c, 0)),
        out_shape=jax.ShapeDtypeStruct((N, N), _f32),
    )(shat)


def _ot_kernel(ot_ref, OT_ref):
    parts = _split3(ot_ref[0])
    row_q = jax.lax.broadcasted_iota(jnp.int32, (128, N), 0)
    col_n = jax.lax.broadcasted_iota(jnp.int32, (128, N), 1)
    OT_ref[...] = jnp.concatenate(
        [_dot3((col_n == 8 * row_q + jj).astype(_bf16), parts)
         for jj in range(8)], axis=1)


def _ot_call(o):
    return pl.pallas_call(
        _ot_kernel,
        in_specs=[pl.BlockSpec((1, N, 16), lambda i: (1, 0, 0))],
        out_specs=pl.BlockSpec((128, 128), lambda i: (0, 0)),
        out_shape=jax.ShapeDtypeStruct((128, 128), _f32),
        grid=(1,),
    )(o)


def _update_kernel(shat_ref, o_ref, OT_ref, M1b_ref, mb1t_ref, M2b_ref,
                   mb2_ref, out_ref):
    macro = pl.program_id(0)
    o_s = o_ref[0]
    ostile = jnp.concatenate([o_s] * 8, axis=1)
    mb1t = mb1t_ref[...]
    parts = []
    for t in range(16):
        OTrow = OT_ref[pl.ds(macro * 16 + t, 1), :]
        D8 = ostile - OTrow
        h8 = jax.nn.relu(_bdot(D8, M1b_ref[...]) + mb1t)
        parts.append(_bdot(h8, M2b_ref[...]))
    out_ref[...] = shat_ref[...] + jnp.concatenate(parts, axis=1) \
        + mb2_ref[0, 0]


def _update_call(shat, o, OT, M1blk, mb1tile, M2blk, mb2_11):
    return pl.pallas_call(
        _update_kernel,
        grid=(8,),
        in_specs=[
            pl.BlockSpec((N, 128), lambda m: (0, m)),
            pl.BlockSpec((1, N, 16), lambda m: (0, 0, 0)),
            pl.BlockSpec((128, 128), lambda m: (0, 0)),
            pl.BlockSpec((128, 128), lambda m: (0, 0)),
            pl.BlockSpec((1, 128), lambda m: (0, 0)),
            pl.BlockSpec((128, 8), lambda m: (0, 0)),
            pl.BlockSpec((1, 1), lambda m: (0, 0)),
        ],
        out_specs=pl.BlockSpec((N, 128), lambda m: (0, m)),
        out_shape=jax.ShapeDtypeStruct((N, N), _f32),
    )(shat, o, OT, M1blk, mb1tile, M2blk, mb2_11)


def kernel(x_s, edge_index_s, edge_attr_s, batch_s, x_t, edge_index_t,
           edge_attr_t, batch_t, W1_self, W1_msg, W1_edge, b1,
           W2_self, W2_msg, W2_edge, b2, M1, mb1, M2, mb2):
    x_all = jnp.stack([x_s[0], x_t[0]])

    dst = jnp.stack([edge_index_s[1], edge_index_t[1]]).astype(jnp.int32)
    src = jnp.stack([edge_index_s[0], edge_index_t[0]]).astype(jnp.int32)
    ea_round = jnp.stack([edge_attr_s, edge_attr_t]).astype(_bf16).astype(_f32)

    A_f32, sea_parts = _build_adj_sc(dst, src, ea_round.transpose(0, 2, 1))
    A = A_f32.astype(_bf16)
    ea1, ea2 = _ea_call(sea_parts, W1_edge, W2_edge)

    h = _gnn_call(x_all, A, ea1, W1_self, W1_msg, b1.reshape(1, -1), 128)
    shat = _shat_init(h, x_s[0, :, 0].reshape(N, 1), x_t[0, :, 0].reshape(1, N))

    rkey = jax.random.key(42)
    r = [jax.random.normal(jax.random.fold_in(rkey, step), (N, 16), _f32)
         for step in range(2)]

    eye8 = jnp.eye(8, dtype=_f32)
    M1blk = jnp.kron(eye8, M1)
    M2blk = jnp.kron(eye8, M2)
    mb1tile = jnp.tile(mb1, (8,)).reshape(1, 128)
    mb2_11 = mb2.reshape(1, 1)

    S0 = None
    for step in range(2):
        S, r_t = _softmax_rt(shat, r[step])
        if step == 0:
            S0 = S
        rst = jnp.stack([r[step], r_t])
        o = _gnn_call(rst, A, ea2, W2_self, W2_msg, b2.reshape(1, -1), 16)
        OT = _ot_call(o)
        shat = _update_call(shat, o, OT, M1blk, mb1tile, M2blk, mb2_11)

    SL = _softmax_call(shat)
    return (S0, SL)


# speedup vs baseline: 2.7668x; 1.0693x over previous
import functools

import jax
import jax.numpy as jnp
from jax.experimental import pallas as pl
from jax.experimental.pallas import tpu as pltpu
from jax.experimental.pallas import tpu_sc as plsc

N = 1024
E = 16384
EB = 1024
NBLK = E // EB
CH = 256
NCH = N // CH

_f32 = jnp.float32
_bf16 = jnp.bfloat16


def _bdot(a, b, dims=None):
    if dims is None:
        dims = (((a.ndim - 1,), (0,)), ((), ()))
    return jax.lax.dot_general(a.astype(_bf16), b.astype(_bf16), dims,
                               preferred_element_type=_f32)


def _split3(v):
    hi = v.astype(_bf16)
    mid = (v - hi.astype(_f32)).astype(_bf16)
    lo = (v - hi.astype(_f32) - mid.astype(_f32)).astype(_bf16)
    return hi, mid, lo


def _dot3(ab, parts):
    d = lambda x: jax.lax.dot_general(ab, x, (((1,), (0,)), ((), ())),
                                      preferred_element_type=_f32)
    return d(parts[0]) + d(parts[1]) + d(parts[2])


_ROWS = 64
_ECH = 1024
_CHK = 2048


def _sc_adj_kernel(dst_hbm, src_hbm, eat_hbm, z_hbm, A_hbm, sea_hbm,
                   dv, sv, ablk, eat, seap):
    g = jax.lax.axis_index("c")
    sid = jax.lax.axis_index("s")
    lo = sid * _ROWS

    ones16 = jnp.ones((16,), _f32)

    pltpu.sync_copy(z_hbm, ablk)
    pltpu.sync_copy(z_hbm.at[pl.ds(0, N * 16)], seap)
    pltpu.sync_copy(dst_hbm.at[g], dv)
    pltpu.sync_copy(src_hbm.at[g], sv)

    def body(i, carry):
        d = dv[pl.ds(i * 16, 16)]
        s_ = sv[pl.ds(i * 16, 16)]
        mask = (d >= lo) & (d < lo + _ROWS)
        flat = jnp.where(mask, (d - lo) * N + s_, 0)
        plsc.addupdate_scatter(ablk, [flat], ones16, mask=mask)
        return carry

    jax.lax.fori_loop(0, E // 16, body, 0)
    pltpu.sync_copy(ablk, A_hbm.at[g, pl.ds(lo * N, _ROWS * N)])

    for half in range(2):
        pltpu.sync_copy(
            eat_hbm.at[g, :, pl.ds(sid * _ECH + half * (_ECH // 2),
                                   _ECH // 2)], eat)

        def ebody(v, carry):
            d16 = dv[pl.ds(sid * _ECH + half * (_ECH // 2) + v * 16, 16)]
            base = d16 * 16
            for m in range(16):
                vals = eat[m, pl.ds(v * 16, 16)]
                plsc.addupdate_scatter(seap, [base + m], vals)
            return carry

        jax.lax.fori_loop(0, _ECH // 2 // 16, ebody, 0)
    pltpu.sync_copy(seap, sea_hbm.at[g, sid])


def _build_adj_sc(dst, src, ea_t):
    mesh = plsc.VectorSubcoreMesh(core_axis_name="c", subcore_axis_name="s")
    run = functools.partial(
        pl.kernel,
        mesh=mesh,
        compiler_params=pltpu.CompilerParams(needs_layout_passes=False),
        out_type=[
            jax.ShapeDtypeStruct((2, N * N), _f32),
            jax.ShapeDtypeStruct((2, 16, N * 16), _f32),
        ],
        scratch_types=[
            pltpu.VMEM((E,), jnp.int32),
            pltpu.VMEM((E,), jnp.int32),
            pltpu.VMEM((_ROWS * N,), _f32),
            pltpu.VMEM((16, _ECH // 2), _f32),
            pltpu.VMEM((N * 16,), _f32),
        ],
    )(_sc_adj_kernel)
    zeros = jnp.zeros((_ROWS * N,), _f32)
    A_flat, sea_parts = run(dst, src, ea_t, zeros)
    return A_flat.reshape(2, N, N), sea_parts.reshape(2, 16, N, 16)


def _adj_build_kernel(dst_ref, src_ref, ea_blk_ref, A_ref, sea_ref):
    j = pl.program_id(1)
    dst = dst_ref[0, 0, 0, :]
    src = src_ref[0, 0, 0, :]
    od = (jax.lax.broadcasted_iota(jnp.int32, (N, EB), 0)
          == dst[None, :]).astype(_bf16)
    os_ = (jax.lax.broadcasted_iota(jnp.int32, (EB, N), 1)
           == src[:, None]).astype(_bf16)
    contrib = jax.lax.dot_general(od, os_, (((1,), (0,)), ((), ())),
                                  preferred_element_type=_f32)
    sea_contrib = jax.lax.dot_general(
        od, ea_blk_ref[0, 0].astype(_bf16), (((1,), (0,)), ((), ())),
        preferred_element_type=_f32)

    @pl.when(j == 0)
    def _():
        A_ref[0] = contrib.astype(_bf16)
        sea_ref[0] = sea_contrib

    @pl.when(j > 0)
    def _():
        A_ref[0] = (A_ref[0].astype(_f32) + contrib).astype(_bf16)
        sea_ref[0] += sea_contrib


def _build_adj(dst_all, src_all, ea_all):
    return pl.pallas_call(
        _adj_build_kernel,
        grid=(2, NBLK),
        in_specs=[
            pl.BlockSpec((1, 1, 1, EB), lambda g, j: (g, j, 0, 0)),
            pl.BlockSpec((1, 1, 1, EB), lambda g, j: (g, j, 0, 0)),
            pl.BlockSpec((1, 1, EB, 16), lambda g, j: (g, j, 0, 0)),
        ],
        out_specs=[
            pl.BlockSpec((1, N, N), lambda g, j: (g, 0, 0)),
            pl.BlockSpec((1, N, 16), lambda g, j: (g, 0, 0)),
        ],
        out_shape=[
            jax.ShapeDtypeStruct((2, N, N), _bf16),
            jax.ShapeDtypeStruct((2, N, 16), _f32),
        ],
    )(dst_all, src_all, ea_all)


def _ea_kernel(sea_ref, W1e_ref, W2e_ref, ea1_ref, ea2_ref):
    parts = _split3(jnp.sum(sea_ref[0], axis=0))
    w1 = W1e_ref[...].astype(_bf16)
    w2 = W2e_ref[...].astype(_bf16)

    def d3(w):
        d = lambda p: jax.lax.dot_general(p, w, (((1,), (0,)), ((), ())),
                                          preferred_element_type=_f32)
        return d(parts[0]) + d(parts[1]) + d(parts[2])

    ea1_ref[0] = d3(w1)
    ea2_ref[0] = d3(w2)


def _ea_call(sea, W1e, W2e):
    return pl.pallas_call(
        _ea_kernel,
        grid=(2,),
        in_specs=[
            pl.BlockSpec((1, 16, N, 16), lambda g: (g, 0, 0, 0)),
            pl.BlockSpec((16, 128), lambda g: (0, 0)),
            pl.BlockSpec((16, 16), lambda g: (0, 0)),
        ],
        out_specs=[
            pl.BlockSpec((1, N, 128), lambda g: (g, 0, 0)),
            pl.BlockSpec((1, N, 16), lambda g: (g, 0, 0)),
        ],
        out_shape=[
            jax.ShapeDtypeStruct((2, N, 128), _f32),
            jax.ShapeDtypeStruct((2, N, 16), _f32),
        ],
    )(sea, W1e, W2e)


def _gnn_kernel(x_ref, A_ref, ea_ref, Wself_ref, Wmsg_ref, b_ref, h_ref):
    c = pl.program_id(1)
    pre_parts = _split3(_bdot(x_ref[0], Wmsg_ref[...]))
    agg = _dot3(A_ref[0], pre_parts)
    xr = x_ref[0, pl.ds(c * CH, CH), :]
    h_ref[0] = jax.nn.relu(_bdot(xr, Wself_ref[...]) + agg
                           + ea_ref[0] + b_ref[...])


def _gnn_call(x_all, A, ea, Wself, Wmsg, b_row, width):
    return pl.pallas_call(
        _gnn_kernel,
        grid=(2, NCH),
        in_specs=[
            pl.BlockSpec((1, N, width), lambda g, c: (g, 0, 0)),
            pl.BlockSpec((1, CH, N), lambda g, c: (g, c, 0)),
            pl.BlockSpec((1, CH, width), lambda g, c: (g, c, 0)),
            pl.BlockSpec((width, width), lambda g, c: (0, 0)),
            pl.BlockSpec((width, width), lambda g, c: (0, 0)),
            pl.BlockSpec((1, width), lambda g, c: (0, 0)),
        ],
        out_specs=pl.BlockSpec((1, CH, width), lambda g, c: (g, c, 0)),
        out_shape=jax.ShapeDtypeStruct((2, N, width), _f32),
    )(x_all, A, ea, Wself, Wmsg, b_row)


def _shat_init_kernel(hs_ref, ht_ref, xs0_ref, xt0_ref, shat_ref):
    shat_ref[...] = (_bdot(hs_ref[0], ht_ref[0], (((1,), (1,)), ((), ())))
                     - 2.0 * (xs0_ref[...] * xt0_ref[...]))


def _shat_init(h, xs0, xt0):
    return pl.pallas_call(
        _shat_init_kernel,
        grid=(NCH,),
        in_specs=[
            pl.BlockSpec((1, CH, 128), lambda c: (0, c, 0)),
            pl.BlockSpec((1, N, 128), lambda c: (1, 0, 0)),
            pl.BlockSpec((CH, 1), lambda c: (c, 0)),
            pl.BlockSpec((1, N), lambda c: (0, 0)),
        ],
        out_specs=pl.BlockSpec((CH, N), lambda c: (c, 0)),
        out_shape=jax.ShapeDtypeStruct((N, N), _f32),
    )(h, h, xs0, xt0)


def _softmax_rt_kernel(shat_ref, r_ref, S_ref, rt_ref):
    c = pl.program_id(0)
    sh = shat_ref[...]
    m = jnp.max(sh, axis=-1, keepdims=True)
    e = jnp.exp(sh - m)
    S = e / jnp.sum(e, axis=-1, keepdims=True)
    S_ref[...] = S
    contrib = _bdot(S, r_ref[...], (((0,), (0,)), ((), ())))

    @pl.when(c == 0)
    def _():
        rt_ref[...] = contrib

    @pl.when(c > 0)
    def _():
        rt_ref[...] += contrib


def _softmax_rt(shat, r_step):
    return pl.pallas_call(
        _softmax_rt_kernel,
        grid=(NCH,),
        in_specs=[
            pl.BlockSpec((CH, N), lambda c: (c, 0)),
            pl.BlockSpec((CH, 16), lambda c: (c, 0)),
        ],
        out_specs=[
            pl.BlockSpec((CH, N), lambda c: (c, 0)),
            pl.BlockSpec((N, 16), lambda c: (0, 0)),
        ],
        out_shape=[
            jax.ShapeDtypeStruct((N, N), _f32),
            jax.ShapeDtypeStruct((N, 16), _f32),
        ],
    )(shat, r_step)


def _softmax_kernel(shat_ref, S_ref):
    sh = shat_ref[...]
    m = jnp.max(sh, axis=-1, keepdims=True)
    e = jnp.exp(sh - m)
    S_ref[...] = e / jnp.sum(e, axis=-1, keepdims=True)


def _softmax_call(shat):
    return pl.pallas_call(
        _softmax_kernel,
        grid=(NCH,),
        in_specs=[pl.BlockSpec((CH, N), lambda c: (c, 0))],
        out_specs=pl.BlockSpec((CH, N), lambda c: (c, 0)),
        out_shape=jax.ShapeDtypeStruct((N, N), _f32),
    )(shat)


def _ot_kernel(ot_ref, OT_ref):
    parts = _split3(ot_ref[0])
    row_q = jax.lax.broadcasted_iota(jnp.int32, (128, N), 0)
    col_n = jax.lax.broadcasted_iota(jnp.int32, (128, N), 1)
    OT_ref[...] = jnp.concatenate(
        [_dot3((col_n == 8 * row_q + jj).astype(_bf16), parts)
         for jj in range(8)], axis=1)


def _ot_call(o):
    return pl.pallas_call(
        _ot_kernel,
        in_specs=[pl.BlockSpec((1, N, 16), lambda i: (1, 0, 0))],
        out_specs=pl.BlockSpec((128, 128), lambda i: (0, 0)),
        out_shape=jax.ShapeDtypeStruct((128, 128), _f32),
        grid=(1,),
    )(o)


def _update_kernel(shat_ref, o_ref, OT_ref, M1b_ref, mb1t_ref, M2b_ref,
                   mb2_ref, out_ref):
    macro = pl.program_id(0)
    o_s = o_ref[0]
    ostile = jnp.concatenate([o_s] * 8, axis=1)
    mb1t = mb1t_ref[...]
    parts = []
    for t in range(16):
        OTrow = OT_ref[pl.ds(macro * 16 + t, 1), :]
        D8 = ostile - OTrow
        h8 = jax.nn.relu(_bdot(D8, M1b_ref[...]) + mb1t)
        parts.append(_bdot(h8, M2b_ref[...]))
    out_ref[...] = shat_ref[...] + jnp.concatenate(parts, axis=1) \
        + mb2_ref[0, 0]


def _update_call(shat, o, OT, M1blk, mb1tile, M2blk, mb2_11):
    return pl.pallas_call(
        _update_kernel,
        grid=(8,),
        in_specs=[
            pl.BlockSpec((N, 128), lambda m: (0, m)),
            pl.BlockSpec((1, N, 16), lambda m: (0, 0, 0)),
            pl.BlockSpec((128, 128), lambda m: (0, 0)),
            pl.BlockSpec((128, 128), lambda m: (0, 0)),
            pl.BlockSpec((1, 128), lambda m: (0, 0)),
            pl.BlockSpec((128, 8), lambda m: (0, 0)),
            pl.BlockSpec((1, 1), lambda m: (0, 0)),
        ],
        out_specs=pl.BlockSpec((N, 128), lambda m: (0, m)),
        out_shape=jax.ShapeDtypeStruct((N, N), _f32),
    )(shat, o, OT, M1blk, mb1tile, M2blk, mb2_11)


def kernel(x_s, edge_index_s, edge_attr_s, batch_s, x_t, edge_index_t,
           edge_attr_t, batch_t, W1_self, W1_msg, W1_edge, b1,
           W2_self, W2_msg, W2_edge, b2, M1, mb1, M2, mb2):
    x_all = jnp.stack([x_s[0], x_t[0]])

    dst = jnp.stack([edge_index_s[1], edge_index_t[1]]).astype(jnp.int32)
    src = jnp.stack([edge_index_s[0], edge_index_t[0]]).astype(jnp.int32)
    ea_round = jnp.stack([edge_attr_s, edge_attr_t]).astype(_bf16).astype(_f32)

    A_f32, sea_parts = _build_adj_sc(dst, src, ea_round.transpose(0, 2, 1))
    A = A_f32.astype(_bf16)
    ea1, ea2 = _ea_call(sea_parts, W1_edge, W2_edge)

    h = _gnn_call(x_all, A, ea1, W1_self, W1_msg, b1.reshape(1, -1), 128)
    shat = _shat_init(h, x_s[0, :, 0].reshape(N, 1), x_t[0, :, 0].reshape(1, N))

    rkey = jax.random.key(42)
    r = [jax.random.normal(jax.random.fold_in(rkey, step), (N, 16), _f32)
         for step in range(2)]

    eye8 = jnp.eye(8, dtype=_f32)
    M1blk = jnp.kron(eye8, M1)
    M2blk = jnp.kron(eye8, M2)
    mb1tile = jnp.tile(mb1, (8,)).reshape(1, 128)
    mb2_11 = mb2.reshape(1, 1)

    S0 = None
    for step in range(2):
        S, r_t = _softmax_rt(shat, r[step])
        if step == 0:
            S0 = S
        rst = jnp.stack([r[step], r_t])
        o = _gnn_call(rst, A, ea2, W2_self, W2_msg, b2.reshape(1, -1), 16)
        OT = _ot_call(o)
        shat = _update_call(shat, o, OT, M1blk, mb1tile, M2blk, mb2_11)

    SL = _softmax_call(shat)
    return (S0, SL)
